# baseline (speedup 1.0000x reference)
import jax
import jax.numpy as jnp
from jax.experimental import pallas as pl
from jax.experimental.pallas import tpu as pltpu

_ROWS = 16384
_COLS = 64
_VR = 8192
_VC = 128
_BS = 1024


def _mul_body(u_ref, v_ref, o_ref):
    o_ref[...] = u_ref[...] * v_ref[...]


@jax.jit
def kernel(user_emb, item_emb):
    u = user_emb.reshape(_VR, _VC)
    v = item_emb.reshape(_VR, _VC)
    spec = pl.BlockSpec((_BS, _VC), lambda i: (i, 0))
    out = pl.pallas_call(
        _mul_body,
        grid=(_VR // _BS,),
        in_specs=[spec, spec],
        out_specs=spec,
        out_shape=jax.ShapeDtypeStruct((_VR, _VC), jnp.float32),
    )(u, v)
    return out.reshape(_ROWS, _COLS)


# speedup vs baseline: 1.7394x; 1.7394x over previous
import jax
import jax.numpy as jnp
from jax.experimental import pallas as pl
from jax.experimental.pallas import tpu as pltpu

_ROWS = 16384
_COLS = 64


def _mul_body(u_ref, v_ref, o_ref):
    o_ref[...] = u_ref[...] * v_ref[...]


@jax.jit
def kernel(user_emb, item_emb):
    return pl.pallas_call(
        _mul_body,
        out_shape=jax.ShapeDtypeStruct((_ROWS, _COLS), jnp.float32),
    )(user_emb, item_emb)


# speedup vs baseline: 1.7700x; 1.0176x over previous
import jax
import jax.numpy as jnp
from jax.experimental import pallas as pl
from jax.experimental.pallas import tpu as pltpu

_ROWS = 16384
_COLS = 64
_NC = 8
_CH = _ROWS // _NC


def _mul_body(u_hbm, v_hbm, o_hbm, u_v, v_v, o_v, sem_u, sem_v, sem_o):
    cps_u = []
    cps_v = []
    for c in range(_NC):
        sl = pl.ds(c * _CH, _CH)
        cp_u = pltpu.make_async_copy(u_hbm.at[sl], u_v.at[sl], sem_u.at[c])
        cp_v = pltpu.make_async_copy(v_hbm.at[sl], v_v.at[sl], sem_v.at[c])
        cp_u.start()
        cp_v.start()
        cps_u.append(cp_u)
        cps_v.append(cp_v)
    cps_o = []
    for c in range(_NC):
        sl = pl.ds(c * _CH, _CH)
        cps_u[c].wait()
        cps_v[c].wait()
        o_v[sl, :] = u_v[sl, :] * v_v[sl, :]
        cp_o = pltpu.make_async_copy(o_v.at[sl], o_hbm.at[sl], sem_o.at[c])
        cp_o.start()
        cps_o.append(cp_o)
    for c in range(_NC):
        cps_o[c].wait()


@jax.jit
def kernel(user_emb, item_emb):
    any_spec = pl.BlockSpec(memory_space=pl.ANY)
    return pl.pallas_call(
        _mul_body,
        in_specs=[any_spec, any_spec],
        out_specs=any_spec,
        out_shape=jax.ShapeDtypeStruct((_ROWS, _COLS), jnp.float32),
        scratch_shapes=[
            pltpu.VMEM((_ROWS, _COLS), jnp.float32),
            pltpu.VMEM((_ROWS, _COLS), jnp.float32),
            pltpu.VMEM((_ROWS, _COLS), jnp.float32),
            pltpu.SemaphoreType.DMA((_NC,)),
            pltpu.SemaphoreType.DMA((_NC,)),
            pltpu.SemaphoreType.DMA((_NC,)),
        ],
    )(user_emb, item_emb)


# speedup vs baseline: 6.1103x; 3.4522x over previous
import jax
import jax.numpy as jnp
from jax.experimental import pallas as pl
from jax.experimental.pallas import tpu as pltpu

_ROWS = 16384
_COLS = 64
_BC = 2048


def _mul_body(u_ref, v_ref, o_ref):
    o_ref[...] = u_ref[...] * v_ref[...]


@jax.jit
def kernel(user_emb, item_emb):
    u = user_emb.T
    v = item_emb.T
    spec = pl.BlockSpec((_COLS, _BC), lambda i: (0, i))
    out = pl.pallas_call(
        _mul_body,
        grid=(_ROWS // _BC,),
        in_specs=[spec, spec],
        out_specs=spec,
        out_shape=jax.ShapeDtypeStruct((_COLS, _ROWS), jnp.float32),
    )(u, v)
    return out.T


# speedup vs baseline: 9.4385x; 1.5447x over previous
import jax
import jax.numpy as jnp
from jax.experimental import pallas as pl
from jax.experimental.pallas import tpu as pltpu

_ROWS = 16384
_COLS = 64
_NC = 8
_CW = _ROWS // _NC


def _mul_body(u_hbm, v_hbm, o_hbm, u_v, v_v, o_v, sem_u, sem_v, sem_o):
    cps_u = []
    cps_v = []
    for c in range(_NC):
        sl = pl.ds(c * _CW, _CW)
        cp_u = pltpu.make_async_copy(u_hbm.at[:, sl], u_v.at[:, sl], sem_u.at[c])
        cp_v = pltpu.make_async_copy(v_hbm.at[:, sl], v_v.at[:, sl], sem_v.at[c])
        cp_u.start()
        cp_v.start()
        cps_u.append(cp_u)
        cps_v.append(cp_v)
    cps_o = []
    for c in range(_NC):
        sl = pl.ds(c * _CW, _CW)
        cps_u[c].wait()
        cps_v[c].wait()
        o_v[:, sl] = u_v[:, sl] * v_v[:, sl]
        cp_o = pltpu.make_async_copy(o_v.at[:, sl], o_hbm.at[:, sl], sem_o.at[c])
        cp_o.start()
        cps_o.append(cp_o)
    for c in range(_NC):
        cps_o[c].wait()


@jax.jit
def kernel(user_emb, item_emb):
    u = user_emb.T
    v = item_emb.T
    any_spec = pl.BlockSpec(memory_space=pl.ANY)
    out = pl.pallas_call(
        _mul_body,
        in_specs=[any_spec, any_spec],
        out_specs=any_spec,
        out_shape=jax.ShapeDtypeStruct((_COLS, _ROWS), jnp.float32),
        scratch_shapes=[
            pltpu.VMEM((_COLS, _ROWS), jnp.float32),
            pltpu.VMEM((_COLS, _ROWS), jnp.float32),
            pltpu.VMEM((_COLS, _ROWS), jnp.float32),
            pltpu.SemaphoreType.DMA((_NC,)),
            pltpu.SemaphoreType.DMA((_NC,)),
            pltpu.SemaphoreType.DMA((_NC,)),
        ],
    )(u, v)
    return out.T
